# baseline (speedup 1.0000x reference)
import jax
import jax.numpy as jnp
from jax.experimental import pallas as pl


def _prelu_body(x_ref, a_ref, o_ref):
    x = x_ref[...]
    a = a_ref[0, 0]
    o_ref[...] = jnp.where(x >= 0, x, a * x)


def kernel(batch_ids, batch_adj_arr, ent_embed, feature_dropout, W, b, w_atten_r, prelu_a):
    B = batch_ids.shape[0]
    row = jnp.concatenate([batch_adj_arr[0], jnp.arange(B, dtype=jnp.int32)])
    col = jnp.concatenate([batch_adj_arr[1], batch_ids.astype(jnp.int32)])
    gat = ent_embed @ W + b
    w1 = w_atten_r[:128, 0]
    w2 = w_atten_r[128:, 0]
    al = gat @ w1
    ar = gat @ w2
    s = al[row] + ar[col]
    att = jnp.exp(-jnp.where(s >= 0, s, 0.2 * s))
    rowsum = jax.ops.segment_sum(att, row, num_segments=B)
    num = jax.ops.segment_sum(att[:, None] * gat[col], row, num_segments=B)
    e_out = num / rowsum[:, None]
    return pl.pallas_call(
        _prelu_body,
        out_shape=jax.ShapeDtypeStruct((B, 128), jnp.float32),
        grid=(10,),
        in_specs=[
            pl.BlockSpec((B // 10, 128), lambda i: (i, 0)),
            pl.BlockSpec((1, 1), lambda i: (0, 0)),
        ],
        out_specs=pl.BlockSpec((B // 10, 128), lambda i: (i, 0)),
    )(e_out, jnp.float32(prelu_a).reshape(1, 1))


# speedup vs baseline: 14.5835x; 14.5835x over previous
import functools

import jax
import jax.numpy as jnp
from jax import lax
from jax.experimental import pallas as pl
from jax.experimental.pallas import tpu as pltpu
from jax.experimental.pallas import tpu_sc as plsc

B = 10000
D = 128
DA = 144
B_PAD = 10240
NC, NS = 2, 16
NW = NC * NS
CHUNK = 128
E_TOT = 320000 + B
CH_PER_W = -(-E_TOT // (NW * CHUNK))
IDX_GRP = 9
TOTAL_CH = CH_PER_W * NW
E_PAD = TOTAL_CH * CHUNK
B_ACC = 10240
ROWS_PER_TILE = B_ACC // NS
ZROWS = 128
R_BLK = 1024
F_BLK = 1000


def _prep_body(ent_ref, waug_ref, baug_ref, wv1_ref, tab_ref, al_ref):
    i = pl.program_id(0)
    x = ent_ref[...]
    y = jnp.dot(x, waug_ref[...], preferred_element_type=jnp.float32) + baug_ref[...]
    rowid = i * R_BLK + lax.broadcasted_iota(jnp.int32, (R_BLK, 1), 0)
    colid = lax.broadcasted_iota(jnp.int32, (1, DA), 1)
    sent = jnp.where(colid == 129, jnp.float32(1e9), jnp.float32(0.0))
    tab_ref[...] = jnp.where(rowid >= B, sent, y)
    al_ref[...] = jnp.sum(x * wv1_ref[...], axis=1).reshape(1, 1, R_BLK)


def _fin_body(p_ref, a_ref, o_ref):
    p = p_ref[...]
    num = p[0, :, :D] + p[1, :, :D]
    den = p[0, :, D:D + 1] + p[1, :, D:D + 1]
    o = num / den
    a = a_ref[0, 0]
    o_ref[...] = jnp.where(o >= 0, o, a * o)


def _edge_body(tab_hbm, al_hbm, row_hbm, col_hbm, out_hbm,
               row_v, col_v, al_v, rows_v, att_v, acc_sh, sem):
    c = lax.axis_index("c")
    s = lax.axis_index("s")
    w = c * NS + s

    def zrow(j, _):
        for g in range(DA // 16):
            rows_v[j, pl.ds(g * 16, 16)] = jnp.zeros((16,), jnp.float32)
        return 0
    lax.fori_loop(0, ZROWS, zrow, 0)
    base = s * ROWS_PER_TILE
    for t in range(ROWS_PER_TILE // ZROWS):
        pltpu.sync_copy(rows_v, acc_sh.at[pl.ds(base + t * ZROWS, ZROWS)])

    pltpu.sync_copy(al_hbm, al_v)

    plsc.subcore_barrier()

    def grp_body(gidx, _):
        pltpu.sync_copy(row_hbm.at[w, pl.ds(gidx * IDX_GRP, IDX_GRP)], row_v)
        pltpu.sync_copy(col_hbm.at[w, pl.ds(gidx * IDX_GRP, IDX_GRP)], col_v)

        def chunk_body(k, _):
            col_sl = col_v.at[k]
            row_sl = row_v.at[k]
            pltpu.async_copy(tab_hbm.at[col_sl], rows_v, sem).wait()
            for g in range(CHUNK // 16):
                rv = row_v[k, pl.ds(g * 16, 16)]
                alv = plsc.load_gather(al_v, [rv])
                ei = lax.iota(jnp.int32, 16) + g * 16
                arv = plsc.load_gather(rows_v, [ei, jnp.full((16,), D + 1, jnp.int32)])
                sc = alv + arv
                lk = jnp.where(sc >= 0, sc, jnp.float32(0.2) * sc)
                att_v[pl.ds(g * 16, 16)] = jnp.exp(-lk)

            def scale_body(e, _):
                a = plsc.load_gather(att_v, [jnp.broadcast_to(e, (16,))])
                for j in range(DA // 16):
                    rows_v[e, pl.ds(j * 16, 16)] = rows_v[e, pl.ds(j * 16, 16)] * a
                return 0
            lax.fori_loop(0, CHUNK, scale_body, 0)
            pltpu.sync_copy(rows_v, acc_sh.at[row_sl], add=True)
            return 0
        lax.fori_loop(0, IDX_GRP, chunk_body, 0)
        return 0
    lax.fori_loop(0, CH_PER_W // IDX_GRP, grp_body, 0)

    plsc.subcore_barrier()
    pltpu.sync_copy(acc_sh.at[pl.ds(base, ROWS_PER_TILE)],
                    out_hbm.at[c, pl.ds(base, ROWS_PER_TILE)])


_edge_kernel = functools.partial(
    pl.kernel,
    _edge_body,
    out_type=jax.ShapeDtypeStruct((NC, B_ACC, DA), jnp.float32),
    mesh=plsc.VectorSubcoreMesh(core_axis_name="c", subcore_axis_name="s"),
    compiler_params=pltpu.CompilerParams(
        needs_layout_passes=False, use_tc_tiling_on_sc=False),
    scratch_types=[
        pltpu.VMEM((IDX_GRP, CHUNK), jnp.int32),
        pltpu.VMEM((IDX_GRP, CHUNK), jnp.int32),
        pltpu.VMEM((B_PAD,), jnp.float32),
        pltpu.VMEM((CHUNK, DA), jnp.float32),
        pltpu.VMEM((CHUNK,), jnp.float32),
        pltpu.VMEM_SHARED((B_ACC, DA), jnp.float32),
        pltpu.SemaphoreType.DMA,
    ],
)()


def kernel(batch_ids, batch_adj_arr, ent_embed, feature_dropout, W, b, w_atten_r, prelu_a):
    w1 = w_atten_r[:D, 0]
    w2 = w_atten_r[D:, 0]
    W_aug = (jnp.zeros((D, DA), jnp.float32)
             .at[:, :D].set(W)
             .at[:, D + 1].set(W @ w2))
    b_aug = (jnp.zeros((DA,), jnp.float32)
             .at[:D].set(b)
             .at[D].set(1.0)
             .at[D + 1].set(jnp.dot(b, w2)))
    wv1 = W @ w1
    ent_pad = jnp.zeros((B_PAD, D), jnp.float32).at[:B].set(ent_embed)

    tab, al2 = pl.pallas_call(
        _prep_body,
        out_shape=(
            jax.ShapeDtypeStruct((B_PAD, DA), jnp.float32),
            jax.ShapeDtypeStruct((B_PAD // R_BLK, 1, R_BLK), jnp.float32),
        ),
        grid=(B_PAD // R_BLK,),
        in_specs=[
            pl.BlockSpec((R_BLK, D), lambda i: (i, 0)),
            pl.BlockSpec((D, DA), lambda i: (0, 0)),
            pl.BlockSpec((1, DA), lambda i: (0, 0)),
            pl.BlockSpec((1, D), lambda i: (0, 0)),
        ],
        out_specs=(
            pl.BlockSpec((R_BLK, DA), lambda i: (i, 0)),
            pl.BlockSpec((1, 1, R_BLK), lambda i: (i, 0, 0)),
        ),
    )(ent_pad, W_aug, b_aug[None, :], wv1[None, :])
    al = al2.reshape(B_PAD) + jnp.dot(b, w1)

    row = jnp.concatenate([batch_adj_arr[0], jnp.arange(B, dtype=jnp.int32)])
    col = jnp.concatenate([batch_adj_arr[1], batch_ids.astype(jnp.int32)])
    rowp = jnp.zeros((E_PAD,), jnp.int32).at[:E_TOT].set(row).reshape(NW, CH_PER_W, CHUNK)
    colp = jnp.full((E_PAD,), B, jnp.int32).at[:E_TOT].set(col).reshape(NW, CH_PER_W, CHUNK)

    parts = _edge_kernel(tab, al, rowp, colp)

    return pl.pallas_call(
        _fin_body,
        out_shape=jax.ShapeDtypeStruct((B, D), jnp.float32),
        grid=(B // F_BLK,),
        in_specs=[
            pl.BlockSpec((NC, F_BLK, DA), lambda i: (0, i, 0)),
            pl.BlockSpec((1, 1), lambda i: (0, 0)),
        ],
        out_specs=pl.BlockSpec((F_BLK, D), lambda i: (i, 0)),
    )(parts, jnp.float32(prelu_a).reshape(1, 1))


# speedup vs baseline: 17.4189x; 1.1944x over previous
import functools

import jax
import jax.numpy as jnp
from jax import lax
from jax.experimental import pallas as pl
from jax.experimental.pallas import tpu as pltpu
from jax.experimental.pallas import tpu_sc as plsc

B = 10000
D = 128
DA = 144
B_PAD = 10240
NC, NS = 2, 16
NW = NC * NS
CHUNK = 64
E_TOT = 320000 + B
CH_PER_W = -(-E_TOT // (NW * CHUNK))
IDX_GRP = 18
TOTAL_CH = CH_PER_W * NW
E_PAD = TOTAL_CH * CHUNK
B_ACC = 10240
ROWS_PER_TILE = B_ACC // NS
ZROWS = CHUNK
R_BLK = 1024
F_BLK = 1000


def _prep_body(ent_ref, waug_ref, baug_ref, wv1_ref, tab_ref, al_ref):
    i = pl.program_id(0)
    x = ent_ref[...]
    y = jnp.dot(x, waug_ref[...], preferred_element_type=jnp.float32) + baug_ref[...]
    rowid = i * R_BLK + lax.broadcasted_iota(jnp.int32, (R_BLK, 1), 0)
    colid = lax.broadcasted_iota(jnp.int32, (1, DA), 1)
    sent = jnp.where(colid == 129, jnp.float32(1e9), jnp.float32(0.0))
    tab_ref[...] = jnp.where(rowid >= B, sent, y)
    al_ref[...] = jnp.sum(x * wv1_ref[...], axis=1).reshape(1, 1, R_BLK)


def _fin_body(p_ref, a_ref, o_ref):
    p = p_ref[...]
    num = p[0, :, :D] + p[1, :, :D]
    den = p[0, :, D:D + 1] + p[1, :, D:D + 1]
    o = num / den
    a = a_ref[0, 0]
    o_ref[...] = jnp.where(o >= 0, o, a * o)


def _edge_body(tab_hbm, al_hbm, row_hbm, col_hbm, out_hbm,
               row_v, col_v, al_v, rows0_v, rows1_v, att_v, acc_sh,
               semg0, semg1, sems0, sems1):
    c = lax.axis_index("c")
    s = lax.axis_index("s")
    w = c * NS + s
    rows = (rows0_v, rows1_v)
    semg = (semg0, semg1)
    sems = (sems0, sems1)

    def zrow(j, _):
        for g in range(DA // 16):
            rows0_v[j, pl.ds(g * 16, 16)] = jnp.zeros((16,), jnp.float32)
        return 0
    lax.fori_loop(0, ZROWS, zrow, 0)
    base = s * ROWS_PER_TILE
    for t in range(ROWS_PER_TILE // ZROWS):
        pltpu.sync_copy(rows0_v, acc_sh.at[pl.ds(base + t * ZROWS, ZROWS)])

    pltpu.sync_copy(al_hbm, al_v)

    plsc.subcore_barrier()

    def gather_start(k, b):
        pltpu.async_copy(tab_hbm.at[col_v.at[k]], rows[b], semg[b])

    def gather_wait(k, b):
        pltpu.make_async_copy(tab_hbm.at[col_v.at[k]], rows[b], semg[b]).wait()

    def scatter_start(k, b):
        pltpu.async_copy(rows[b], acc_sh.at[row_v.at[k]], sems[b], add=True)

    def scatter_wait(k, b):
        pltpu.make_async_copy(rows[b], acc_sh.at[row_v.at[k]], sems[b]).wait()

    def compute(k, b):
        rbuf = rows[b]
        for g in range(CHUNK // 16):
            rv = row_v[k, pl.ds(g * 16, 16)]
            alv = plsc.load_gather(al_v, [rv])
            ei = lax.iota(jnp.int32, 16) + g * 16
            arv = plsc.load_gather(rbuf, [ei, jnp.full((16,), D + 1, jnp.int32)])
            sc = alv + arv
            lk = jnp.where(sc >= 0, sc, jnp.float32(0.2) * sc)
            att_v[pl.ds(g * 16, 16)] = jnp.exp(-lk)

        def scale_body(e, _):
            a = plsc.load_gather(att_v, [jnp.broadcast_to(e, (16,))])
            for j in range(DA // 16):
                rbuf[e, pl.ds(j * 16, 16)] = rbuf[e, pl.ds(j * 16, 16)] * a
            return 0
        lax.fori_loop(0, CHUNK, scale_body, 0)

    def grp_body(gidx, _):
        pltpu.sync_copy(row_hbm.at[w, pl.ds(gidx * IDX_GRP, IDX_GRP)], row_v)
        pltpu.sync_copy(col_hbm.at[w, pl.ds(gidx * IDX_GRP, IDX_GRP)], col_v)

        gather_start(0, 0)

        def pair_body(p, _):
            k = p * 2

            @pl.when(p > 0)
            def _():
                scatter_wait(k - 1, 1)
            gather_start(k + 1, 1)

            gather_wait(k, 0)
            compute(k, 0)
            scatter_start(k, 0)

            gather_wait(k + 1, 1)
            compute(k + 1, 1)
            scatter_start(k + 1, 1)

            scatter_wait(k, 0)

            @pl.when(k + 2 < IDX_GRP)
            def _():
                gather_start(k + 2, 0)
            return 0
        lax.fori_loop(0, IDX_GRP // 2, pair_body, 0)
        scatter_wait(IDX_GRP - 1, 1)
        return 0
    lax.fori_loop(0, CH_PER_W // IDX_GRP, grp_body, 0)

    plsc.subcore_barrier()
    pltpu.sync_copy(acc_sh.at[pl.ds(base, ROWS_PER_TILE)],
                    out_hbm.at[c, pl.ds(base, ROWS_PER_TILE)])


_edge_kernel = functools.partial(
    pl.kernel,
    _edge_body,
    out_type=jax.ShapeDtypeStruct((NC, B_ACC, DA), jnp.float32),
    mesh=plsc.VectorSubcoreMesh(core_axis_name="c", subcore_axis_name="s"),
    compiler_params=pltpu.CompilerParams(
        needs_layout_passes=False, use_tc_tiling_on_sc=False),
    scratch_types=[
        pltpu.VMEM((IDX_GRP, CHUNK), jnp.int32),
        pltpu.VMEM((IDX_GRP, CHUNK), jnp.int32),
        pltpu.VMEM((B_PAD,), jnp.float32),
        pltpu.VMEM((CHUNK, DA), jnp.float32),
        pltpu.VMEM((CHUNK, DA), jnp.float32),
        pltpu.VMEM((CHUNK,), jnp.float32),
        pltpu.VMEM_SHARED((B_ACC, DA), jnp.float32),
        pltpu.SemaphoreType.DMA,
        pltpu.SemaphoreType.DMA,
        pltpu.SemaphoreType.DMA,
        pltpu.SemaphoreType.DMA,
    ],
)()


def kernel(batch_ids, batch_adj_arr, ent_embed, feature_dropout, W, b, w_atten_r, prelu_a):
    w1 = w_atten_r[:D, 0]
    w2 = w_atten_r[D:, 0]
    W_aug = (jnp.zeros((D, DA), jnp.float32)
             .at[:, :D].set(W)
             .at[:, D + 1].set(W @ w2))
    b_aug = (jnp.zeros((DA,), jnp.float32)
             .at[:D].set(b)
             .at[D].set(1.0)
             .at[D + 1].set(jnp.dot(b, w2)))
    wv1 = W @ w1
    ent_pad = jnp.zeros((B_PAD, D), jnp.float32).at[:B].set(ent_embed)

    tab, al2 = pl.pallas_call(
        _prep_body,
        out_shape=(
            jax.ShapeDtypeStruct((B_PAD, DA), jnp.float32),
            jax.ShapeDtypeStruct((B_PAD // R_BLK, 1, R_BLK), jnp.float32),
        ),
        grid=(B_PAD // R_BLK,),
        in_specs=[
            pl.BlockSpec((R_BLK, D), lambda i: (i, 0)),
            pl.BlockSpec((D, DA), lambda i: (0, 0)),
            pl.BlockSpec((1, DA), lambda i: (0, 0)),
            pl.BlockSpec((1, D), lambda i: (0, 0)),
        ],
        out_specs=(
            pl.BlockSpec((R_BLK, DA), lambda i: (i, 0)),
            pl.BlockSpec((1, 1, R_BLK), lambda i: (i, 0, 0)),
        ),
    )(ent_pad, W_aug, b_aug[None, :], wv1[None, :])
    al = al2.reshape(B_PAD) + jnp.dot(b, w1)

    row = jnp.concatenate([batch_adj_arr[0], jnp.arange(B, dtype=jnp.int32)])
    col = jnp.concatenate([batch_adj_arr[1], batch_ids.astype(jnp.int32)])
    rowp = jnp.zeros((E_PAD,), jnp.int32).at[:E_TOT].set(row).reshape(NW, CH_PER_W, CHUNK)
    colp = jnp.full((E_PAD,), B, jnp.int32).at[:E_TOT].set(col).reshape(NW, CH_PER_W, CHUNK)

    parts = _edge_kernel(tab, al, rowp, colp)

    return pl.pallas_call(
        _fin_body,
        out_shape=jax.ShapeDtypeStruct((B, D), jnp.float32),
        grid=(B // F_BLK,),
        in_specs=[
            pl.BlockSpec((NC, F_BLK, DA), lambda i: (0, i, 0)),
            pl.BlockSpec((1, 1), lambda i: (0, 0)),
        ],
        out_specs=pl.BlockSpec((F_BLK, D), lambda i: (i, 0)),
    )(parts, jnp.float32(prelu_a).reshape(1, 1))


# speedup vs baseline: 19.1027x; 1.0967x over previous
import functools

import jax
import jax.numpy as jnp
from jax import lax
from jax.experimental import pallas as pl
from jax.experimental.pallas import tpu as pltpu
from jax.experimental.pallas import tpu_sc as plsc

B = 10000
D = 128
DA = 144
B_PAD = 10240
NC, NS = 2, 16
NW = NC * NS
CHUNK = 64
E_TOT = 320000 + B
CH_PER_W = -(-E_TOT // (NW * CHUNK))
IDX_GRP = 18
TOTAL_CH = CH_PER_W * NW
E_PAD = TOTAL_CH * CHUNK
B_ACC = 10240
ROWS_PER_TILE = B_ACC // NS
ZROWS = CHUNK
R_BLK = 1024
F_BLK = 1000


def _prep_body(ent_ref, waug_ref, baug_ref, wv1_ref, tab_ref, al_ref):
    i = pl.program_id(0)
    x = ent_ref[...]
    y = jnp.dot(x, waug_ref[...], preferred_element_type=jnp.float32) + baug_ref[...]
    rowid = i * R_BLK + lax.broadcasted_iota(jnp.int32, (R_BLK, 1), 0)
    colid = lax.broadcasted_iota(jnp.int32, (1, DA), 1)
    sent = jnp.where(colid == 129, jnp.float32(1e9), jnp.float32(0.0))
    tab_ref[...] = jnp.where(rowid >= B, sent, y)
    al_ref[...] = jnp.sum(x * wv1_ref[...], axis=1).reshape(1, 1, R_BLK)


def _fin_body(p_ref, a_ref, o_ref):
    p = p_ref[...]
    num = p[0, :, :D] + p[1, :, :D]
    den = p[0, :, D:D + 1] + p[1, :, D:D + 1]
    o = num / den
    a = a_ref[0, 0]
    o_ref[...] = jnp.where(o >= 0, o, a * o)


def _edge_body(tab_hbm, al_hbm, row_hbm, col_hbm, out_hbm,
               row_v, col_v, al_v, rows0_v, rows1_v, att_v, acc_sh,
               semg0, semg1, sems0, sems1):
    c = lax.axis_index("c")
    s = lax.axis_index("s")
    w = c * NS + s
    rows = (rows0_v, rows1_v)
    semg = (semg0, semg1)
    sems = (sems0, sems1)

    def zrow(j, _):
        for g in range(DA // 16):
            rows0_v[j, pl.ds(g * 16, 16)] = jnp.zeros((16,), jnp.float32)
        return 0
    lax.fori_loop(0, ZROWS, zrow, 0)
    base = s * ROWS_PER_TILE
    for t in range(ROWS_PER_TILE // ZROWS):
        pltpu.sync_copy(rows0_v, acc_sh.at[pl.ds(base + t * ZROWS, ZROWS)])

    pltpu.sync_copy(al_hbm, al_v)

    plsc.subcore_barrier()

    def gather_start(k, b):
        pltpu.async_copy(tab_hbm.at[col_v.at[k]], rows[b], semg[b])

    def gather_wait(k, b):
        pltpu.make_async_copy(tab_hbm.at[col_v.at[k]], rows[b], semg[b]).wait()

    def scatter_start(k, b):
        pltpu.async_copy(rows[b], acc_sh.at[row_v.at[k]], sems[b], add=True)

    def scatter_wait(k, b):
        pltpu.make_async_copy(rows[b], acc_sh.at[row_v.at[k]], sems[b]).wait()

    def compute(k, b):
        rbuf = rows[b]
        for g in range(CHUNK // 16):
            rv = row_v[k, pl.ds(g * 16, 16)]
            alv = plsc.load_gather(al_v, [rv])
            ei = lax.iota(jnp.int32, 16) + g * 16
            arv = plsc.load_gather(rbuf, [ei, jnp.full((16,), D + 1, jnp.int32)])
            sc = alv + arv
            lk = jnp.where(sc >= 0, sc, jnp.float32(0.2) * sc)
            att_v[pl.ds(g * 16, 16)] = jnp.exp(-lk)

        @plsc.parallel_loop(0, CHUNK, unroll=8)
        def scale_body(e):
            a = plsc.load_gather(att_v, [jnp.broadcast_to(e, (16,))])
            for j in range(DA // 16):
                rbuf[e, pl.ds(j * 16, 16)] = rbuf[e, pl.ds(j * 16, 16)] * a

    def grp_body(gidx, _):
        pltpu.sync_copy(row_hbm.at[w, pl.ds(gidx * IDX_GRP, IDX_GRP)], row_v)
        pltpu.sync_copy(col_hbm.at[w, pl.ds(gidx * IDX_GRP, IDX_GRP)], col_v)

        gather_start(0, 0)

        def pair_body(p, _):
            k = p * 2

            @pl.when(p > 0)
            def _():
                scatter_wait(k - 1, 1)
            gather_start(k + 1, 1)

            gather_wait(k, 0)
            compute(k, 0)
            scatter_start(k, 0)

            gather_wait(k + 1, 1)
            compute(k + 1, 1)
            scatter_start(k + 1, 1)

            scatter_wait(k, 0)

            @pl.when(k + 2 < IDX_GRP)
            def _():
                gather_start(k + 2, 0)
            return 0
        lax.fori_loop(0, IDX_GRP // 2, pair_body, 0)
        scatter_wait(IDX_GRP - 1, 1)
        return 0
    lax.fori_loop(0, CH_PER_W // IDX_GRP, grp_body, 0)

    plsc.subcore_barrier()
    pltpu.sync_copy(acc_sh.at[pl.ds(base, ROWS_PER_TILE)],
                    out_hbm.at[c, pl.ds(base, ROWS_PER_TILE)])


_edge_kernel = functools.partial(
    pl.kernel,
    _edge_body,
    out_type=jax.ShapeDtypeStruct((NC, B_ACC, DA), jnp.float32),
    mesh=plsc.VectorSubcoreMesh(core_axis_name="c", subcore_axis_name="s"),
    compiler_params=pltpu.CompilerParams(
        needs_layout_passes=False, use_tc_tiling_on_sc=False),
    scratch_types=[
        pltpu.VMEM((IDX_GRP, CHUNK), jnp.int32),
        pltpu.VMEM((IDX_GRP, CHUNK), jnp.int32),
        pltpu.VMEM((B_PAD,), jnp.float32),
        pltpu.VMEM((CHUNK, DA), jnp.float32),
        pltpu.VMEM((CHUNK, DA), jnp.float32),
        pltpu.VMEM((CHUNK,), jnp.float32),
        pltpu.VMEM_SHARED((B_ACC, DA), jnp.float32),
        pltpu.SemaphoreType.DMA,
        pltpu.SemaphoreType.DMA,
        pltpu.SemaphoreType.DMA,
        pltpu.SemaphoreType.DMA,
    ],
)()


def kernel(batch_ids, batch_adj_arr, ent_embed, feature_dropout, W, b, w_atten_r, prelu_a):
    w1 = w_atten_r[:D, 0]
    w2 = w_atten_r[D:, 0]
    W_aug = (jnp.zeros((D, DA), jnp.float32)
             .at[:, :D].set(W)
             .at[:, D + 1].set(W @ w2))
    b_aug = (jnp.zeros((DA,), jnp.float32)
             .at[:D].set(b)
             .at[D].set(1.0)
             .at[D + 1].set(jnp.dot(b, w2)))
    wv1 = W @ w1
    ent_pad = jnp.zeros((B_PAD, D), jnp.float32).at[:B].set(ent_embed)

    tab, al2 = pl.pallas_call(
        _prep_body,
        out_shape=(
            jax.ShapeDtypeStruct((B_PAD, DA), jnp.float32),
            jax.ShapeDtypeStruct((B_PAD // R_BLK, 1, R_BLK), jnp.float32),
        ),
        grid=(B_PAD // R_BLK,),
        in_specs=[
            pl.BlockSpec((R_BLK, D), lambda i: (i, 0)),
            pl.BlockSpec((D, DA), lambda i: (0, 0)),
            pl.BlockSpec((1, DA), lambda i: (0, 0)),
            pl.BlockSpec((1, D), lambda i: (0, 0)),
        ],
        out_specs=(
            pl.BlockSpec((R_BLK, DA), lambda i: (i, 0)),
            pl.BlockSpec((1, 1, R_BLK), lambda i: (i, 0, 0)),
        ),
    )(ent_pad, W_aug, b_aug[None, :], wv1[None, :])
    al = al2.reshape(B_PAD) + jnp.dot(b, w1)

    row = jnp.concatenate([batch_adj_arr[0], jnp.arange(B, dtype=jnp.int32)])
    col = jnp.concatenate([batch_adj_arr[1], batch_ids.astype(jnp.int32)])
    rowp = jnp.zeros((E_PAD,), jnp.int32).at[:E_TOT].set(row).reshape(NW, CH_PER_W, CHUNK)
    colp = jnp.full((E_PAD,), B, jnp.int32).at[:E_TOT].set(col).reshape(NW, CH_PER_W, CHUNK)

    parts = _edge_kernel(tab, al, rowp, colp)

    return pl.pallas_call(
        _fin_body,
        out_shape=jax.ShapeDtypeStruct((B, D), jnp.float32),
        grid=(B // F_BLK,),
        in_specs=[
            pl.BlockSpec((NC, F_BLK, DA), lambda i: (0, i, 0)),
            pl.BlockSpec((1, 1), lambda i: (0, 0)),
        ],
        out_specs=pl.BlockSpec((F_BLK, D), lambda i: (i, 0)),
    )(parts, jnp.float32(prelu_a).reshape(1, 1))
